# baseline (speedup 1.0000x reference)
import functools
import math

import jax
import jax.numpy as jnp
from jax import lax
from jax.experimental import pallas as pl
from jax.experimental.pallas import tpu as pltpu
from jax.experimental.pallas import tpu_sc as plsc

_NC = 2
_NS = 16
_NW = _NC * _NS
_LANES = 128
_K = 8


def _make_gather(V, D, B):
    assert B % (_NW * _LANES) == 0
    rows_idx = B // _LANES
    per_w = rows_idx // _NW
    k = _K
    assert per_w % (2 * k) == 0
    nchunks = per_w // k
    half = nchunks // 2
    C = k * _LANES
    mesh = plsc.VectorSubcoreMesh(core_axis_name="c", subcore_axis_name="s")

    @functools.partial(
        pl.kernel,
        mesh=mesh,
        compiler_params=pltpu.CompilerParams(use_tc_tiling_on_sc=False),
        out_type=jax.ShapeDtypeStruct((B, D), jnp.bfloat16),
        scratch_types=[
            pltpu.VMEM((k, _LANES), jnp.int32),
            pltpu.VMEM((k, _LANES), jnp.int32),
            pltpu.VMEM((C, D), jnp.bfloat16),
            pltpu.VMEM((C, D), jnp.bfloat16),
            pltpu.SemaphoreType.DMA,
            pltpu.SemaphoreType.DMA,
            pltpu.SemaphoreType.DMA,
            pltpu.SemaphoreType.DMA,
        ],
    )
    def gath(table_hbm, idx_hbm, out_hbm, idx0, idx1, rows0, rows1,
             g0, g1, s0, s1):
        wid = lax.axis_index("s") * _NC + lax.axis_index("c")
        idxs = (idx0, idx1)
        rowss = (rows0, rows1)
        gsem = (g0, g1)
        ssem = (s0, s1)

        def fire_gather(g, b):
            row0 = wid * per_w + g * k
            pltpu.sync_copy(idx_hbm.at[pl.ds(row0, k)], idxs[b])
            for j in range(k):
                pltpu.async_copy(
                    table_hbm.at[idxs[b].at[j]],
                    rowss[b].at[pl.ds(j * _LANES, _LANES)],
                    gsem[b],
                )

        def drain_gather(b):
            for j in range(k):
                pltpu.make_async_copy(
                    table_hbm.at[idxs[b].at[j]],
                    rowss[b].at[pl.ds(j * _LANES, _LANES)],
                    gsem[b],
                ).wait()

        def fire_store(g, b):
            base = (wid * per_w + g * k) * _LANES
            pltpu.async_copy(rowss[b], out_hbm.at[pl.ds(base, C)], ssem[b])

        def drain_store(b):
            pltpu.make_async_copy(rowss[b], out_hbm.at[pl.ds(0, C)],
                                  ssem[b]).wait()

        fire_gather(0, 0)

        def body(i, carry):
            c0 = 2 * i
            c1 = c0 + 1

            @pl.when(i > 0)
            def _():
                drain_store(1)

            fire_gather(c1, 1)
            drain_gather(0)
            fire_store(c0, 0)

            @pl.when(i + 1 < half)
            def _():
                drain_store(0)
                fire_gather(c0 + 2, 0)

            drain_gather(1)
            fire_store(c1, 1)
            return carry

        lax.fori_loop(0, half, body, 0)
        drain_store(0)
        drain_store(1)

    return gath


def _scale_convert(pbf, scale, block_rows=8192):
    M, L = pbf.shape
    assert M % block_rows == 0

    def body(i_ref, o_ref):
        o_ref[...] = i_ref[...].astype(jnp.float32) * scale

    return pl.pallas_call(
        body,
        grid=(M // block_rows,),
        in_specs=[pl.BlockSpec((block_rows, L), lambda i: (i, 0))],
        out_specs=pl.BlockSpec((block_rows, L), lambda i: (i, 0)),
        out_shape=jax.ShapeDtypeStruct((M, L), jnp.float32),
    )(pbf)


def kernel(x, table):
    V, D = table.shape
    B = x.size
    scale = float(math.sqrt(float(D)))
    tb = table.astype(jnp.bfloat16)
    xi = x.reshape(-1).astype(jnp.int32).reshape(B // _LANES, _LANES)
    out_bf = _make_gather(V, D, B)(tb, xi)
    out = _scale_convert(out_bf, scale)
    return out.reshape(*x.shape, D)


# speedup vs baseline: 1.4050x; 1.4050x over previous
import functools
import math

import jax
import jax.numpy as jnp
from jax import lax
from jax.experimental import pallas as pl
from jax.experimental.pallas import tpu as pltpu
from jax.experimental.pallas import tpu_sc as plsc

_NC = 2
_NS = 16
_NW = _NC * _NS
_LANES = 128
_K = 8


def _make_gather(V, D, B):
    assert B % (_NW * _LANES) == 0
    rows_idx = B // _LANES
    per_w = rows_idx // _NW
    k = _K
    assert per_w % (2 * k) == 0
    nchunks = per_w // k
    half = nchunks // 2
    C = k * _LANES
    scale = float(math.sqrt(float(D)))
    mesh = plsc.VectorSubcoreMesh(core_axis_name="c", subcore_axis_name="s")

    @functools.partial(
        pl.kernel,
        mesh=mesh,
        compiler_params=pltpu.CompilerParams(use_tc_tiling_on_sc=False),
        out_type=jax.ShapeDtypeStruct((B, D), jnp.float32),
        scratch_types=[
            pltpu.VMEM((k, _LANES), jnp.int32),
            pltpu.VMEM((k, _LANES), jnp.int32),
            pltpu.VMEM((C, D), jnp.float32),
            pltpu.VMEM((C, D), jnp.float32),
            pltpu.SemaphoreType.DMA,
            pltpu.SemaphoreType.DMA,
            pltpu.SemaphoreType.DMA,
            pltpu.SemaphoreType.DMA,
        ],
    )
    def gath(table_hbm, idx_hbm, out_hbm, idx0, idx1, rows0, rows1,
             g0, g1, s0, s1):
        wid = lax.axis_index("s") * _NC + lax.axis_index("c")
        idxs = (idx0, idx1)
        rowss = (rows0, rows1)
        gsem = (g0, g1)
        ssem = (s0, s1)

        def fire_gather(g, b):
            row0 = wid * per_w + g * k
            pltpu.sync_copy(idx_hbm.at[pl.ds(row0, k)], idxs[b])
            for j in range(k):
                pltpu.async_copy(
                    table_hbm.at[idxs[b].at[j]],
                    rowss[b].at[pl.ds(j * _LANES, _LANES)],
                    gsem[b],
                )

        def drain_gather(b):
            for j in range(k):
                pltpu.make_async_copy(
                    table_hbm.at[idxs[b].at[j]],
                    rowss[b].at[pl.ds(j * _LANES, _LANES)],
                    gsem[b],
                ).wait()

        def fire_store(g, b):
            base = (wid * per_w + g * k) * _LANES
            pltpu.async_copy(rowss[b], out_hbm.at[pl.ds(base, C)], ssem[b])

        def drain_store(b):
            pltpu.make_async_copy(rowss[b], out_hbm.at[pl.ds(0, C)],
                                  ssem[b]).wait()

        fire_gather(0, 0)

        def body(i, carry):
            c0 = 2 * i
            c1 = c0 + 1

            @pl.when(i > 0)
            def _():
                drain_store(1)

            fire_gather(c1, 1)
            drain_gather(0)
            fire_store(c0, 0)

            @pl.when(i + 1 < half)
            def _():
                drain_store(0)
                fire_gather(c0 + 2, 0)

            drain_gather(1)
            fire_store(c1, 1)
            return carry

        lax.fori_loop(0, half, body, 0)
        drain_store(0)
        drain_store(1)

    return gath


def _prescale_transpose(tbT, scale, block_cols=8192):
    Dd, V = tbT.shape

    def body(i_ref, o_ref):
        o_ref[...] = i_ref[...].T * scale

    return pl.pallas_call(
        body,
        grid=((V + block_cols - 1) // block_cols,),
        in_specs=[pl.BlockSpec((Dd, block_cols), lambda i: (0, i))],
        out_specs=pl.BlockSpec((block_cols, Dd), lambda i: (i, 0)),
        out_shape=jax.ShapeDtypeStruct((V, Dd), jnp.float32),
    )(tbT)


def kernel(x, table):
    V, D = table.shape
    B = x.size
    scale = float(math.sqrt(float(D)))
    tb = _prescale_transpose(jnp.transpose(table), scale)
    xi = x.reshape(-1).astype(jnp.int32).reshape(B // _LANES, _LANES)
    out = _make_gather(V, D, B)(tb, xi)
    return out.reshape(*x.shape, D)
